# Initial kernel scaffold; baseline (speedup 1.0000x reference)
import jax
import jax.numpy as jnp
from jax.experimental import pallas as pl


def kernel(x, vec, edge_index, edge_rbf, edge_vector, we_t, be, w1_t, b1, w2_t, b2):
    raise NotImplementedError("write your pallas kernel here")


# speedup vs baseline: 1.0094x; 1.0094x over previous
import functools
import math

import jax
import jax.numpy as jnp
from jax.experimental import pallas as pl
from jax.experimental.pallas import tpu as pltpu


def _round_up(v, m):
    return ((v + m - 1) // m) * m


def _fused_kernel(j_ref, i_ref, rbf_ref, ev_ref, x_ref, vecf_ref,
                  w1_ref, b1_ref, w2_ref, b2_ref, we_ref, be_ref,
                  out_ref, xh_ref, *, hidden):
    H = hidden
    te = rbf_ref.shape[0]
    n_nodes = x_ref.shape[0]
    inv_sqrt_3 = 1.0 / math.sqrt(3.0)
    inv_sqrt_h = 1.0 / math.sqrt(float(H))

    @pl.when(pl.program_id(1) == 0)
    def _init():
        h = jnp.dot(x_ref[...], w1_ref[...],
                    preferred_element_type=jnp.float32) + b1_ref[...]
        h = h * jax.nn.sigmoid(h) * (1.0 / 0.6)
        xh_ref[...] = jnp.dot(h, w2_ref[...],
                              preferred_element_type=jnp.float32) + b2_ref[...]
        out_ref[...] = jnp.zeros_like(out_ref)

    jj = j_ref[...]
    ii = i_ref[0]

    gath = (jax.lax.broadcasted_iota(jnp.int32, (te, n_nodes), 1) == jj
            ).astype(jnp.float32)
    scat_t = (jax.lax.broadcasted_iota(jnp.int32, (n_nodes, te), 0) == ii
              ).astype(jnp.float32)

    rbf_h = jnp.dot(rbf_ref[...], we_ref[...],
                    preferred_element_type=jnp.float32) + be_ref[...]

    x_h_j = jnp.dot(gath, xh_ref[...], preferred_element_type=jnp.float32)
    vec_j = jnp.dot(gath, vecf_ref[...], preferred_element_type=jnp.float32)

    xji = x_h_j * rbf_h * inv_sqrt_3
    x1 = xji[:, :H]
    x2 = xji[:, H:2 * H]
    x3 = xji[:, 2 * H:]

    ev = ev_ref[...]
    pieces = [x3]
    for k in range(3):
        vk = (x1 * vec_j[:, k * H:(k + 1) * H] + x2 * ev[:, k:k + 1]) * inv_sqrt_h
        pieces.append(vk)
    msgs = jnp.concatenate(pieces, axis=-1)

    out_ref[0] += jnp.dot(scat_t, msgs, preferred_element_type=jnp.float32)


@functools.partial(jax.jit, static_argnames=("tile_e",))
def _message_passing(x, vec, edge_index, edge_rbf, edge_vector,
                     we_t, be, w1_t, b1, w2_t, b2, *, tile_e=512):
    N, H = x.shape
    E, F = edge_rbf.shape

    te = min(tile_e, _round_up(E, 8))
    e_pad = _round_up(E, 2 * te)
    pe = e_pad - E
    j = edge_index[0].astype(jnp.int32)
    i = edge_index[1].astype(jnp.int32)
    rbf = edge_rbf
    ev = edge_vector
    if pe:
        j = jnp.pad(j, (0, pe))
        i = jnp.pad(i, (0, pe), constant_values=N)
        rbf = jnp.pad(rbf, ((0, pe), (0, 0)))
        ev = jnp.pad(ev, ((0, pe), (0, 0)))
    j2 = j.reshape(e_pad, 1)
    i3 = i.reshape(e_pad // te, 1, te)
    vec_flat = vec.reshape(N, 3 * H)

    tpc = e_pad // (2 * te)

    kern = functools.partial(_fused_kernel, hidden=H)
    d_out = pl.pallas_call(
        kern,
        out_shape=jax.ShapeDtypeStruct((2, N, 4 * H), jnp.float32),
        grid=(2, tpc),
        in_specs=[
            pl.BlockSpec((te, 1), lambda c, s: (c * tpc + s, 0)),
            pl.BlockSpec((1, 1, te), lambda c, s: (c * tpc + s, 0, 0)),
            pl.BlockSpec((te, F), lambda c, s: (c * tpc + s, 0)),
            pl.BlockSpec((te, 3), lambda c, s: (c * tpc + s, 0)),
            pl.BlockSpec((N, H), lambda c, s: (0, 0)),
            pl.BlockSpec((N, 3 * H), lambda c, s: (0, 0)),
            pl.BlockSpec((H, H // 2), lambda c, s: (0, 0)),
            pl.BlockSpec((1, H // 2), lambda c, s: (0, 0)),
            pl.BlockSpec((H // 2, 3 * H), lambda c, s: (0, 0)),
            pl.BlockSpec((1, 3 * H), lambda c, s: (0, 0)),
            pl.BlockSpec((F, 3 * H), lambda c, s: (0, 0)),
            pl.BlockSpec((1, 3 * H), lambda c, s: (0, 0)),
        ],
        out_specs=pl.BlockSpec((1, N, 4 * H), lambda c, s: (c, 0, 0)),
        scratch_shapes=[pltpu.VMEM((N, 3 * H), jnp.float32)],
        compiler_params=pltpu.CompilerParams(
            dimension_semantics=("parallel", "arbitrary"),
            vmem_limit_bytes=64 * 1024 * 1024),
    )(j2, i3, rbf, ev, x, vec_flat,
      w1_t, b1.reshape(1, -1), w2_t, b2.reshape(1, -1), we_t, be.reshape(1, -1))

    d = d_out[0] + d_out[1]
    d_x = d[:, :H]
    d_vec = d[:, H:].reshape(N, 3, H)
    return d_x, d_vec


def kernel(x, vec, edge_index, edge_rbf, edge_vector, we_t, be, w1_t, b1, w2_t, b2):
    return _message_passing(x, vec, edge_index, edge_rbf, edge_vector,
                            we_t, be, w1_t, b1, w2_t, b2)


# speedup vs baseline: 1.1324x; 1.1218x over previous
import functools
import math

import jax
import jax.numpy as jnp
from jax.experimental import pallas as pl
from jax.experimental.pallas import tpu as pltpu


def _round_up(v, m):
    return ((v + m - 1) // m) * m


def _fused_kernel(j_ref, i_ref, rbf_ref, ev_ref, x_ref, vecf_ref,
                  w1_ref, b1_ref, w2_ref, b2_ref, we_ref, be_ref,
                  out_ref, tab_ref, *, hidden):
    H = hidden
    te = rbf_ref.shape[0]
    n_nodes = x_ref.shape[0]

    @pl.when(pl.program_id(0) == 0)
    def _init():
        h = jnp.dot(x_ref[...], w1_ref[...],
                    preferred_element_type=jnp.float32) + b1_ref[...]
        h = h * jax.nn.sigmoid(h) * (1.0 / 0.6)
        tab_ref[:, :3 * H] = jnp.dot(h, w2_ref[...],
                                     preferred_element_type=jnp.float32) + b2_ref[...]
        tab_ref[:, 3 * H:] = vecf_ref[...]
        out_ref[...] = jnp.zeros_like(out_ref)

    jj = j_ref[...]
    ii = i_ref[0]

    gath = (jax.lax.broadcasted_iota(jnp.int32, (te, n_nodes), 1) == jj
            ).astype(jnp.float32)
    scat_t = (jax.lax.broadcasted_iota(jnp.int32, (n_nodes, te), 0) == ii
              ).astype(jnp.float32)

    rbf_h = jnp.dot(rbf_ref[...], we_ref[...],
                    preferred_element_type=jnp.float32) + be_ref[...]

    gathered = jnp.dot(gath, tab_ref[...], preferred_element_type=jnp.float32)

    xji = gathered[:, :3 * H] * rbf_h
    x1 = xji[:, :H]
    x2 = xji[:, H:2 * H]
    x3 = xji[:, 2 * H:]

    ev = ev_ref[...]
    pieces = [x3]
    for k in range(3):
        vk = x1 * gathered[:, (3 + k) * H:(4 + k) * H] + x2 * ev[:, k:k + 1]
        pieces.append(vk)
    msgs = jnp.concatenate(pieces, axis=-1)

    out_ref[...] += jnp.dot(scat_t, msgs, preferred_element_type=jnp.float32)


@functools.partial(jax.jit, static_argnames=("tile_e",))
def _message_passing(x, vec, edge_index, edge_rbf, edge_vector,
                     we_t, be, w1_t, b1, w2_t, b2, *, tile_e=512):
    N, H = x.shape
    E, F = edge_rbf.shape

    te = min(tile_e, _round_up(E, 8))
    e_pad = _round_up(E, te)
    pe = e_pad - E
    j = edge_index[0].astype(jnp.int32)
    i = edge_index[1].astype(jnp.int32)
    rbf = edge_rbf
    ev = edge_vector
    if pe:
        j = jnp.pad(j, (0, pe))
        i = jnp.pad(i, (0, pe), constant_values=N)
        rbf = jnp.pad(rbf, ((0, pe), (0, 0)))
        ev = jnp.pad(ev, ((0, pe), (0, 0)))
    j2 = j.reshape(e_pad, 1)
    i3 = i.reshape(e_pad // te, 1, te)
    vec_flat = vec.reshape(N, 3 * H)

    inv3 = 1.0 / math.sqrt(3.0)
    invh = 1.0 / math.sqrt(float(H))
    col_scale = jnp.concatenate([
        jnp.full((2 * H,), inv3 * invh, jnp.float32),
        jnp.full((H,), inv3, jnp.float32)])
    we_s = we_t * col_scale[None, :]
    be_s = (be * col_scale).reshape(1, -1)

    kern = functools.partial(_fused_kernel, hidden=H)
    d_out = pl.pallas_call(
        kern,
        out_shape=jax.ShapeDtypeStruct((N, 4 * H), jnp.float32),
        grid=(e_pad // te,),
        in_specs=[
            pl.BlockSpec((te, 1), lambda s: (s, 0)),
            pl.BlockSpec((1, 1, te), lambda s: (s, 0, 0)),
            pl.BlockSpec((te, F), lambda s: (s, 0)),
            pl.BlockSpec((te, 3), lambda s: (s, 0)),
            pl.BlockSpec((N, H), lambda s: (0, 0)),
            pl.BlockSpec((N, 3 * H), lambda s: (0, 0)),
            pl.BlockSpec((H, H // 2), lambda s: (0, 0)),
            pl.BlockSpec((1, H // 2), lambda s: (0, 0)),
            pl.BlockSpec((H // 2, 3 * H), lambda s: (0, 0)),
            pl.BlockSpec((1, 3 * H), lambda s: (0, 0)),
            pl.BlockSpec((F, 3 * H), lambda s: (0, 0)),
            pl.BlockSpec((1, 3 * H), lambda s: (0, 0)),
        ],
        out_specs=pl.BlockSpec((N, 4 * H), lambda s: (0, 0)),
        scratch_shapes=[pltpu.VMEM((N, 6 * H), jnp.float32)],
        compiler_params=pltpu.CompilerParams(
            dimension_semantics=("arbitrary",),
            vmem_limit_bytes=64 * 1024 * 1024),
    )(j2, i3, rbf, ev, x, vec_flat,
      w1_t, b1.reshape(1, -1), w2_t, b2.reshape(1, -1), we_s, be_s)

    d_x = d_out[:, :H]
    d_vec = d_out[:, H:].reshape(N, 3, H)
    return d_x, d_vec


def kernel(x, vec, edge_index, edge_rbf, edge_vector, we_t, be, w1_t, b1, w2_t, b2):
    return _message_passing(x, vec, edge_index, edge_rbf, edge_vector,
                            we_t, be, w1_t, b1, w2_t, b2)


# speedup vs baseline: 1.2135x; 1.0716x over previous
import functools
import math

import jax
import jax.numpy as jnp
from jax.experimental import pallas as pl
from jax.experimental.pallas import tpu as pltpu


def _round_up(v, m):
    return ((v + m - 1) // m) * m


def _fused_kernel(j_ref, i_ref, rbf_ref, ev_ref, x_ref, vecf_ref,
                  w1_ref, b1_ref, w2_ref, b2_ref, we_ref, be_ref,
                  out_ref, tab_ref, *, hidden):
    H = hidden
    te = rbf_ref.shape[0]
    n_nodes = x_ref.shape[0]

    @pl.when(pl.program_id(0) == 0)
    def _init():
        h = jnp.dot(x_ref[...], w1_ref[...],
                    preferred_element_type=jnp.float32) + b1_ref[...]
        h = h * jax.nn.sigmoid(h) * (1.0 / 0.6)
        tab_ref[:, :3 * H] = jnp.dot(h, w2_ref[...],
                                     preferred_element_type=jnp.float32) + b2_ref[...]
        tab_ref[:, 3 * H:] = vecf_ref[...]
        out_ref[...] = jnp.zeros_like(out_ref)

    jj = j_ref[...]
    ii = i_ref[0]

    gath = (jax.lax.broadcasted_iota(jnp.int32, (te, n_nodes), 1) == jj
            ).astype(jnp.float32)
    scat_t = (jax.lax.broadcasted_iota(jnp.int32, (n_nodes, te), 0) == ii
              ).astype(jnp.float32)

    rbf_h = jnp.dot(rbf_ref[...], we_ref[...],
                    preferred_element_type=jnp.float32) + be_ref[...]

    gathered = jnp.dot(gath, tab_ref[...], preferred_element_type=jnp.float32)

    xji = gathered[:, :3 * H] * rbf_h
    x1 = xji[:, :H]
    x2 = xji[:, H:2 * H]
    x3 = xji[:, 2 * H:]

    ev = ev_ref[...]
    pieces = [x3]
    for k in range(3):
        vk = x1 * gathered[:, (3 + k) * H:(4 + k) * H] + x2 * ev[:, k:k + 1]
        pieces.append(vk)
    msgs = jnp.concatenate(pieces, axis=-1)

    out_ref[...] += jnp.dot(scat_t, msgs, preferred_element_type=jnp.float32)


@functools.partial(jax.jit, static_argnames=("tile_e",))
def _message_passing(x, vec, edge_index, edge_rbf, edge_vector,
                     we_t, be, w1_t, b1, w2_t, b2, *, tile_e=1024):
    N, H = x.shape
    E, F = edge_rbf.shape

    te = min(tile_e, _round_up(E, 8))
    e_pad = _round_up(E, te)
    pe = e_pad - E
    j = edge_index[0].astype(jnp.int32)
    i = edge_index[1].astype(jnp.int32)
    rbf = edge_rbf
    ev = edge_vector
    if pe:
        j = jnp.pad(j, (0, pe))
        i = jnp.pad(i, (0, pe), constant_values=N)
        rbf = jnp.pad(rbf, ((0, pe), (0, 0)))
        ev = jnp.pad(ev, ((0, pe), (0, 0)))
    j2 = j.reshape(e_pad, 1)
    i3 = i.reshape(e_pad // te, 1, te)
    vec_flat = vec.reshape(N, 3 * H)

    inv3 = 1.0 / math.sqrt(3.0)
    invh = 1.0 / math.sqrt(float(H))
    col_scale = jnp.concatenate([
        jnp.full((2 * H,), inv3 * invh, jnp.float32),
        jnp.full((H,), inv3, jnp.float32)])
    we_s = we_t * col_scale[None, :]
    be_s = (be * col_scale).reshape(1, -1)

    kern = functools.partial(_fused_kernel, hidden=H)
    d_out = pl.pallas_call(
        kern,
        out_shape=jax.ShapeDtypeStruct((N, 4 * H), jnp.float32),
        grid=(e_pad // te,),
        in_specs=[
            pl.BlockSpec((te, 1), lambda s: (s, 0)),
            pl.BlockSpec((1, 1, te), lambda s: (s, 0, 0)),
            pl.BlockSpec((te, F), lambda s: (s, 0)),
            pl.BlockSpec((te, 3), lambda s: (s, 0)),
            pl.BlockSpec((N, H), lambda s: (0, 0)),
            pl.BlockSpec((N, 3 * H), lambda s: (0, 0)),
            pl.BlockSpec((H, H // 2), lambda s: (0, 0)),
            pl.BlockSpec((1, H // 2), lambda s: (0, 0)),
            pl.BlockSpec((H // 2, 3 * H), lambda s: (0, 0)),
            pl.BlockSpec((1, 3 * H), lambda s: (0, 0)),
            pl.BlockSpec((F, 3 * H), lambda s: (0, 0)),
            pl.BlockSpec((1, 3 * H), lambda s: (0, 0)),
        ],
        out_specs=pl.BlockSpec((N, 4 * H), lambda s: (0, 0)),
        scratch_shapes=[pltpu.VMEM((N, 6 * H), jnp.float32)],
        compiler_params=pltpu.CompilerParams(
            dimension_semantics=("arbitrary",),
            vmem_limit_bytes=64 * 1024 * 1024),
    )(j2, i3, rbf, ev, x, vec_flat,
      w1_t, b1.reshape(1, -1), w2_t, b2.reshape(1, -1), we_s, be_s)

    d_x = d_out[:, :H]
    d_vec = d_out[:, H:].reshape(N, 3, H)
    return d_x, d_vec


def kernel(x, vec, edge_index, edge_rbf, edge_vector, we_t, be, w1_t, b1, w2_t, b2):
    return _message_passing(x, vec, edge_index, edge_rbf, edge_vector,
                            we_t, be, w1_t, b1, w2_t, b2)
